# baseline (speedup 1.0000x reference)
import functools

import jax
import jax.numpy as jnp
from jax import lax
from jax.experimental import pallas as pl
from jax.experimental.pallas import tpu as pltpu
from jax.experimental.pallas import tpu_sc as plsc

VOCAB = 49408
HIDDEN = 768
SEQ = 77
BATCH = 4096

ROWS = BATCH * SEQ
NW = 32
ROWS_PER_W = ROWS // NW
K = 32
CHUNKS_PER_W = ROWS_PER_W // K
LANES = 16
NSL = HIDDEN // LANES

_mesh = plsc.VectorSubcoreMesh(core_axis_name="c", subcore_axis_name="s")


@functools.partial(
    pl.kernel,
    out_type=jax.ShapeDtypeStruct((ROWS, HIDDEN), jnp.float32),
    mesh=_mesh,
    scratch_types=[
        pltpu.VMEM((CHUNKS_PER_W, K), jnp.int32),
        pltpu.VMEM((SEQ, HIDDEN), jnp.float32),
        pltpu.VMEM((K, HIDDEN), jnp.float32),
        pltpu.SemaphoreType.DMA,
    ],
)
def _embed_sc(ids_hbm, table_hbm, pos_hbm, out_hbm, idx_v, pos_v, buf, sem):
    wid = lax.axis_index("s") * 2 + lax.axis_index("c")
    pltpu.sync_copy(ids_hbm.at[wid], idx_v)
    pltpu.sync_copy(pos_hbm, pos_v)

    def chunk_body(c, p):
        pltpu.async_copy(table_hbm.at[idx_v.at[c]], buf, sem).wait()

        def row_body(i, p):
            for j in range(NSL):
                sl = pl.ds(j * LANES, LANES)
                buf[i, sl] = buf[i, sl] + pos_v[p, sl]
            p = p + 1
            return jnp.where(p == SEQ, 0, p)

        p = lax.fori_loop(0, K, row_body, p)
        pltpu.sync_copy(buf, out_hbm.at[pl.ds(wid * ROWS_PER_W + c * K, K)])
        return p

    lax.fori_loop(0, CHUNKS_PER_W, chunk_body, jnp.int32(0))


_MASK_BB = 32


def _mask_body(o_ref):
    r = lax.broadcasted_iota(jnp.int32, (SEQ, SEQ), 0)
    c = lax.broadcasted_iota(jnp.int32, (SEQ, SEQ), 1)
    m = jnp.where(c <= r, jnp.float32(0), jnp.finfo(jnp.float32).min)
    o_ref[...] = jnp.broadcast_to(m[None, None], o_ref.shape)


def _make_mask():
    return pl.pallas_call(
        _mask_body,
        grid=(BATCH // _MASK_BB,),
        out_specs=pl.BlockSpec((_MASK_BB, 1, SEQ, SEQ), lambda i: (i, 0, 0, 0)),
        out_shape=jax.ShapeDtypeStruct((BATCH, 1, SEQ, SEQ), jnp.float32),
    )()


def kernel(input_ids, attention_mask, image_embeds, token_embedding, position_embedding):
    ids3d = input_ids.astype(jnp.int32).reshape(NW, CHUNKS_PER_W, K)
    hidden = _embed_sc(ids3d, token_embedding, position_embedding)
    hidden = hidden.reshape(BATCH, SEQ, HIDDEN)
    causal = _make_mask()
    return (input_ids, attention_mask, hidden, causal, image_embeds)


# speedup vs baseline: 1.3516x; 1.3516x over previous
import functools

import jax
import jax.numpy as jnp
from jax import lax
from jax.experimental import pallas as pl
from jax.experimental.pallas import tpu as pltpu
from jax.experimental.pallas import tpu_sc as plsc

VOCAB = 49408
HIDDEN = 768
SEQ = 77
BATCH = 4096

ROWS = BATCH * SEQ
NW = 32
ROWS_PER_W = ROWS // NW
K = 32
CHUNKS_PER_W = ROWS_PER_W // K
NPAIR = CHUNKS_PER_W // 2
LANES = 16
NSL = HIDDEN // LANES

_mesh = plsc.VectorSubcoreMesh(core_axis_name="c", subcore_axis_name="s")


@functools.partial(
    pl.kernel,
    out_type=jax.ShapeDtypeStruct((ROWS, HIDDEN), jnp.float32),
    mesh=_mesh,
    scratch_types=[
        pltpu.VMEM((ROWS_PER_W,), jnp.int32),
        pltpu.VMEM((SEQ, HIDDEN), jnp.float32),
        pltpu.VMEM((K, HIDDEN), jnp.float32),
        pltpu.VMEM((K, HIDDEN), jnp.float32),
        pltpu.SemaphoreType.DMA,
        pltpu.SemaphoreType.DMA,
        pltpu.SemaphoreType.DMA,
        pltpu.SemaphoreType.DMA,
    ],
)
def _embed_sc(ids_hbm, table_hbm, pos_hbm, out_hbm, idx_v, pos_v,
              buf_a, buf_b, gs_a, gs_b, ss_a, ss_b):
    wid = lax.axis_index("s") * 2 + lax.axis_index("c")
    pltpu.sync_copy(ids_hbm.at[pl.ds(wid * ROWS_PER_W, ROWS_PER_W)], idx_v)
    pltpu.sync_copy(pos_hbm, pos_v)
    out_base = wid * ROWS_PER_W

    def start_gather(c, buf, sem):
        pltpu.async_copy(table_hbm.at[idx_v.at[pl.ds(c * K, K)]], buf, sem)

    def wait_gather(c, buf, sem):
        pltpu.make_async_copy(table_hbm.at[idx_v.at[pl.ds(c * K, K)]], buf, sem).wait()

    def start_store(c, buf, sem):
        pltpu.async_copy(buf, out_hbm.at[pl.ds(out_base + c * K, K)], sem)

    def wait_store(c, buf, sem):
        pltpu.make_async_copy(buf, out_hbm.at[pl.ds(out_base + c * K, K)], sem).wait()

    def add_pos(buf, p):
        def row_body(i, p):
            for j in range(NSL):
                sl = pl.ds(j * LANES, LANES)
                plsc.addupdate(buf.at[i, sl], pos_v[p, sl])
            p = p + 1
            return jnp.where(p == SEQ, 0, p)
        return lax.fori_loop(0, K, row_body, p)

    start_gather(0, buf_a, gs_a)

    def pair_body(t, p):
        c_a = 2 * t
        c_b = c_a + 1
        wait_gather(c_a, buf_a, gs_a)

        @pl.when(t > 0)
        def _():
            wait_store(c_b - 2, buf_b, ss_b)

        start_gather(c_b, buf_b, gs_b)
        p = add_pos(buf_a, p)
        start_store(c_a, buf_a, ss_a)
        wait_gather(c_b, buf_b, gs_b)

        @pl.when(t < NPAIR - 1)
        def _():
            wait_store(c_a, buf_a, ss_a)
            start_gather(c_a + 2, buf_a, gs_a)

        p = add_pos(buf_b, p)
        start_store(c_b, buf_b, ss_b)
        return p

    lax.fori_loop(0, NPAIR, pair_body, jnp.int32(0))
    wait_store(CHUNKS_PER_W - 2, buf_a, ss_a)
    wait_store(CHUNKS_PER_W - 1, buf_b, ss_b)


_MASK_BB = 32


def _mask_body(o_ref):
    r = lax.broadcasted_iota(jnp.int32, (SEQ, SEQ), 0)
    c = lax.broadcasted_iota(jnp.int32, (SEQ, SEQ), 1)
    m = jnp.where(c <= r, jnp.float32(0), jnp.finfo(jnp.float32).min)
    o_ref[...] = jnp.broadcast_to(m[None, None], o_ref.shape)


def _make_mask():
    return pl.pallas_call(
        _mask_body,
        grid=(BATCH // _MASK_BB,),
        out_specs=pl.BlockSpec((_MASK_BB, 1, SEQ, SEQ), lambda i: (i, 0, 0, 0)),
        out_shape=jax.ShapeDtypeStruct((BATCH, 1, SEQ, SEQ), jnp.float32),
    )()


def kernel(input_ids, attention_mask, image_embeds, token_embedding, position_embedding):
    ids_flat = input_ids.astype(jnp.int32).reshape(ROWS)
    hidden = _embed_sc(ids_flat, token_embedding, position_embedding)
    hidden = hidden.reshape(BATCH, SEQ, HIDDEN)
    causal = _make_mask()
    return (input_ids, attention_mask, hidden, causal, image_embeds)


# speedup vs baseline: 2.3468x; 1.7363x over previous
import functools

import jax
import jax.numpy as jnp
from jax import lax
from jax.experimental import pallas as pl
from jax.experimental.pallas import tpu as pltpu
from jax.experimental.pallas import tpu_sc as plsc

VOCAB = 49408
HIDDEN = 768
SEQ = 77
BATCH = 4096

NW = 32
BPW = BATCH // NW
K = 32
BLKS = BPW // K
CHUNKS_PER_W = SEQ * BLKS
NPAIR = CHUNKS_PER_W // 2
IDS_PER_W = SEQ * BPW
LANES = 16
NSL = HIDDEN // LANES

_mesh = plsc.VectorSubcoreMesh(core_axis_name="c", subcore_axis_name="s")


@functools.partial(
    pl.kernel,
    out_type=jax.ShapeDtypeStruct((SEQ, BATCH, HIDDEN), jnp.float32),
    mesh=_mesh,
    scratch_types=[
        pltpu.VMEM((IDS_PER_W,), jnp.int32),
        pltpu.VMEM((SEQ * HIDDEN,), jnp.float32),
        pltpu.VMEM((K, HIDDEN), jnp.float32),
        pltpu.VMEM((K, HIDDEN), jnp.float32),
        pltpu.SemaphoreType.DMA,
        pltpu.SemaphoreType.DMA,
        pltpu.SemaphoreType.DMA,
        pltpu.SemaphoreType.DMA,
    ],
)
def _embed_sc(ids_hbm, table_hbm, pos_hbm, out_hbm, idx_v, pos_v,
              buf_a, buf_b, gs_a, gs_b, ss_a, ss_b):
    wid = lax.axis_index("s") * 2 + lax.axis_index("c")
    pltpu.sync_copy(ids_hbm.at[pl.ds(wid * IDS_PER_W, IDS_PER_W)], idx_v)
    pltpu.sync_copy(pos_hbm, pos_v)
    b_base = wid * BPW

    def start_gather(c, buf, sem):
        pltpu.async_copy(table_hbm.at[idx_v.at[pl.ds(c * K, K)]], buf, sem)

    def wait_gather(c, buf, sem):
        pltpu.make_async_copy(table_hbm.at[idx_v.at[pl.ds(c * K, K)]], buf, sem).wait()

    def dst(c):
        return out_hbm.at[c // BLKS, pl.ds(b_base + (c % BLKS) * K, K)]

    def start_store(c, buf, sem):
        pltpu.async_copy(buf, dst(c), sem)

    def wait_store(c, buf, sem):
        pltpu.make_async_copy(buf, dst(c), sem).wait()

    def add_pos(buf, c):
        p_base = (c // BLKS) * HIDDEN

        def row_body(i, carry):
            for j in range(NSL):
                plsc.addupdate(buf.at[i, pl.ds(j * LANES, LANES)],
                               pos_v[pl.ds(p_base + j * LANES, LANES)])
            return carry
        lax.fori_loop(0, K, row_body, jnp.int32(0))

    start_gather(0, buf_a, gs_a)

    def pair_body(t, carry):
        c_a = 2 * t
        c_b = c_a + 1
        wait_gather(c_a, buf_a, gs_a)

        @pl.when(t > 0)
        def _():
            wait_store(c_b - 2, buf_b, ss_b)

        start_gather(c_b, buf_b, gs_b)
        add_pos(buf_a, c_a)
        start_store(c_a, buf_a, ss_a)
        wait_gather(c_b, buf_b, gs_b)

        @pl.when(t < NPAIR - 1)
        def _():
            wait_store(c_a, buf_a, ss_a)
            start_gather(c_a + 2, buf_a, gs_a)

        add_pos(buf_b, c_b)
        start_store(c_b, buf_b, ss_b)
        return carry

    lax.fori_loop(0, NPAIR, pair_body, jnp.int32(0))
    wait_store(CHUNKS_PER_W - 2, buf_a, ss_a)
    wait_store(CHUNKS_PER_W - 1, buf_b, ss_b)


_MASK_BB = 32


def _mask_body(o_ref):
    r = lax.broadcasted_iota(jnp.int32, (SEQ, SEQ), 0)
    c = lax.broadcasted_iota(jnp.int32, (SEQ, SEQ), 1)
    m = jnp.where(c <= r, jnp.float32(0), jnp.finfo(jnp.float32).min)
    o_ref[...] = jnp.broadcast_to(m[None, None], o_ref.shape)


def _make_mask():
    return pl.pallas_call(
        _mask_body,
        grid=(BATCH // _MASK_BB,),
        out_specs=pl.BlockSpec((_MASK_BB, 1, SEQ, SEQ), lambda i: (i, 0, 0, 0)),
        out_shape=jax.ShapeDtypeStruct((BATCH, 1, SEQ, SEQ), jnp.float32),
    )()


def kernel(input_ids, attention_mask, image_embeds, token_embedding, position_embedding):
    ids_prep = (input_ids.astype(jnp.int32)
                .reshape(NW, BPW, SEQ).transpose(0, 2, 1).reshape(-1))
    out_t = _embed_sc(ids_prep, token_embedding, position_embedding.reshape(-1))
    hidden = jnp.transpose(out_t, (1, 0, 2))
    causal = _make_mask()
    return (input_ids, attention_mask, hidden, causal, image_embeds)
